# Initial kernel scaffold; baseline (speedup 1.0000x reference)
import jax
import jax.numpy as jnp
from jax.experimental import pallas as pl


def kernel(x, edge_index, edge_attr, W1, b1, g1, be1, W2, b2, g2, be2):
    raise NotImplementedError("write your pallas kernel here")


# speedup vs baseline: 9.1917x; 9.1917x over previous
import functools

import jax
import jax.numpy as jnp
from jax import lax
from jax.experimental import pallas as pl
from jax.experimental.pallas import tpu as pltpu
from jax.experimental.pallas import tpu_sc as plsc

N = 10000
E = 320000
D = 128

NC = 2
NS = 16
L = 16
TILES = NC * NS

CH = 128
E2 = E + N
EPT = 10368
NCHUNK = EPT // CH
E_PAD = EPT * TILES
NP = 10240
RPT = NP // NS

def _deg_body(dst_hbm, w_hbm, out_hbm, deg_l, idx_b, w_b):
    c = lax.axis_index("c")
    s = lax.axis_index("s")
    wid = c * NS + s

    def zbody(i, _):
        deg_l[pl.ds(i * L, L)] = jnp.zeros((L,), jnp.float32)
        return 0

    lax.fori_loop(0, N // L, zbody, 0)

    base = wid * EPT

    def chunk_body(ci, _):
        off = base + ci * CH
        pltpu.sync_copy(dst_hbm.at[pl.ds(off, CH)], idx_b)
        pltpu.sync_copy(w_hbm.at[pl.ds(off, CH)], w_b)
        for j in range(CH // L):
            sl = pl.ds(j * L, L)
            plsc.addupdate_scatter(deg_l, [idx_b[sl]], w_b[sl])
        return 0

    lax.fori_loop(0, NCHUNK, chunk_body, 0)
    pltpu.sync_copy(deg_l, out_hbm.at[wid])


def _agg_body(src_hbm, dst_hbm, w_hbm, dinv_hbm, h_hbm, zeros_hbm, out_hbm,
                dinv_l, srcb, dstb, wb, normb, rows, acc_sh, sem):
    c = lax.axis_index("c")
    s = lax.axis_index("s")
    wid = c * NS + s

    pltpu.sync_copy(dinv_hbm, dinv_l)
    pltpu.sync_copy(zeros_hbm.at[pl.ds(s * RPT, RPT)],
                    acc_sh.at[pl.ds(s * RPT, RPT)])
    plsc.subcore_barrier()

    base = wid * EPT

    def chunk_body(ci, _):
        off = base + ci * CH
        pltpu.sync_copy(src_hbm.at[pl.ds(off, CH)], srcb)
        pltpu.sync_copy(dst_hbm.at[pl.ds(off, CH)], dstb)
        pltpu.sync_copy(w_hbm.at[pl.ds(off, CH)], wb)
        pltpu.async_copy(h_hbm.at[srcb], rows, sem).wait()
        for j in range(CH // L):
            sl = pl.ds(j * L, L)
            dsv = plsc.load_gather(dinv_l, [srcb[sl]])
            ddv = plsc.load_gather(dinv_l, [dstb[sl]])
            normb[sl] = dsv * ddv * wb[sl]

        def scale_body(e, _):
            nv = plsc.load_gather(normb, [jnp.full((L,), e, jnp.int32)])
            for j in range(D // L):
                sl2 = pl.ds(j * L, L)
                rows[e, sl2] = rows[e, sl2] * nv
            return 0

        lax.fori_loop(0, CH, scale_body, 0)
        pltpu.sync_copy(rows, acc_sh.at[dstb], add=True)
        return 0

    lax.fori_loop(0, NCHUNK, chunk_body, 0)
    plsc.subcore_barrier()
    pltpu.sync_copy(acc_sh.at[pl.ds(s * RPT, RPT)],
                    out_hbm.at[c, pl.ds(s * RPT, RPT)])


@functools.lru_cache(maxsize=None)
def _sc_kernels():
    mesh = plsc.VectorSubcoreMesh(core_axis_name="c", subcore_axis_name="s")
    deg_k = pl.kernel(
        _deg_body,
        out_type=jax.ShapeDtypeStruct((TILES, N), jnp.float32),
        mesh=mesh,
        scratch_types=[
            pltpu.VMEM((N,), jnp.float32),
            pltpu.VMEM((CH,), jnp.int32),
            pltpu.VMEM((CH,), jnp.float32),
        ],
        compiler_params=pltpu.CompilerParams(needs_layout_passes=False),
    )
    agg_k = pl.kernel(
        _agg_body,
        out_type=jax.ShapeDtypeStruct((NC, NP, D), jnp.float32),
        mesh=mesh,
        scratch_types=[
            pltpu.VMEM((N,), jnp.float32),
            pltpu.VMEM((CH,), jnp.int32),
            pltpu.VMEM((CH,), jnp.int32),
            pltpu.VMEM((CH,), jnp.float32),
            pltpu.VMEM((CH,), jnp.float32),
            pltpu.VMEM((CH, D), jnp.float32),
            pltpu.VMEM_SHARED((NP, D), jnp.float32),
            pltpu.SemaphoreType.DMA,
        ],
        compiler_params=pltpu.CompilerParams(needs_layout_passes=False),
    )
    return deg_k, agg_k


def _tc1_body(degp_ref, x_ref, w1_ref, dinv_ref, h1_ref):
    deg = jnp.sum(degp_ref[...], axis=0, keepdims=True)
    dinv_ref[...] = jnp.where(deg > 0, lax.rsqrt(deg), 0.0)
    h1_ref[...] = jnp.dot(x_ref[...], w1_ref[...],
                          preferred_element_type=jnp.float32)


def _bn_relu(accp_ref, b_ref, g_ref, be_ref):
    a = accp_ref[0, :N, :] + accp_ref[1, :N, :] + b_ref[...]
    a = jnp.maximum(a, 0.0)
    m = jnp.mean(a, axis=0, keepdims=True)
    v = jnp.maximum(jnp.mean(a * a, axis=0, keepdims=True) - m * m, 0.0)
    return g_ref[...] * (a - m) * lax.rsqrt(v + 1e-5) + be_ref[...]


def _tc_mid_body(accp_ref, b_ref, g_ref, be_ref, w2_ref, h2_ref):
    h = _bn_relu(accp_ref, b_ref, g_ref, be_ref)
    h2_ref[...] = jnp.dot(h, w2_ref[...], preferred_element_type=jnp.float32)


def _tc_post_body(accp_ref, b_ref, g_ref, be_ref, out_ref):
    out_ref[...] = _bn_relu(accp_ref, b_ref, g_ref, be_ref)


_tc1 = pl.pallas_call(
    _tc1_body,
    out_shape=[jax.ShapeDtypeStruct((1, N), jnp.float32),
               jax.ShapeDtypeStruct((N, D), jnp.float32)],
)

_tc_mid = pl.pallas_call(
    _tc_mid_body,
    out_shape=jax.ShapeDtypeStruct((N, D), jnp.float32),
)

_tc_post = pl.pallas_call(
    _tc_post_body,
    out_shape=jax.ShapeDtypeStruct((N, D), jnp.float32),
)


def kernel(x, edge_index, edge_attr, W1, b1, g1, be1, W2, b2, g2, be2):
    src = edge_index[0]
    dst = edge_index[1]
    loop = jnp.arange(N, dtype=jnp.int32)
    padn = E_PAD - E2
    zi = jnp.zeros((padn,), jnp.int32)
    src2 = jnp.concatenate([src, loop, zi])
    dst2 = jnp.concatenate([dst, loop, zi])
    w2_ = jnp.concatenate([edge_attr, jnp.ones((N,), jnp.float32),
                           jnp.zeros((padn,), jnp.float32)])

    deg_k, agg_k = _sc_kernels()
    degp = deg_k(dst2, w2_)
    dinv2d, h1 = _tc1(degp, x, W1)
    dinv = dinv2d.reshape(N)
    zeros_nd = jnp.zeros((NP, D), jnp.float32)

    accp1 = agg_k(src2, dst2, w2_, dinv, h1, zeros_nd)
    h2 = _tc_mid(accp1, b1.reshape(1, D), g1.reshape(1, D),
                 be1.reshape(1, D), W2)
    accp2 = agg_k(src2, dst2, w2_, dinv, h2, zeros_nd)
    out = _tc_post(accp2, b2.reshape(1, D), g2.reshape(1, D),
                   be2.reshape(1, D))
    return out
